# Initial kernel scaffold; baseline (speedup 1.0000x reference)
import jax
import jax.numpy as jnp
from jax.experimental import pallas as pl


def kernel(h, edge_index, Ws1, Wn1, b1, Ws2, Wn2, b2, Ws3, Wn3, b3, bn1g, bn1b, bn1m, bn1v, bn2g, bn2b, bn2m, bn2v, bn3g, bn3b, bn3m, bn3v, fc1W, fc1b, fc2W, fc2b, fc3W, fc3b):
    raise NotImplementedError("write your pallas kernel here")


# speedup vs baseline: 4.1059x; 4.1059x over previous
import functools

import jax
import jax.numpy as jnp
from jax import lax
from jax.experimental import pallas as pl
from jax.experimental.pallas import tpu as pltpu
from jax.experimental.pallas import tpu_sc as plsc

_N = 10000
_NPAD = 10240
_E = 320000
_R = 256
_NT = _NPAD // _R
_K = 128
_NSUB = 16
_NCORE = 2
_H = 256


def _seg_sum_body(xsplit, srcr, dstr, zeros, out, srcv, dstv, idxv, rows, acc, gsem):
    c = lax.axis_index("c")
    s = lax.axis_index("s")
    rps = _NPAD // _NSUB
    pltpu.sync_copy(zeros, acc.at[pl.ds(s * rps, rps)])
    plsc.subcore_barrier()
    coff = c * _NPAD
    nchunk = _E // _K
    base_n = nchunk // _NSUB
    rem = nchunk - base_n * _NSUB
    nch = base_n + jnp.where(s < rem, 1, 0)

    def body(t, carry):
        base = (s + t * _NSUB) * _K
        pltpu.sync_copy(srcr.at[pl.ds(base, _K)], srcv)
        pltpu.sync_copy(dstr.at[pl.ds(base, _K)], dstv)
        for g in range(_K // 16):
            idxv[pl.ds(g * 16, 16)] = srcv[pl.ds(g * 16, 16)] + coff
        pltpu.async_copy(xsplit.at[idxv], rows, gsem).wait()
        pltpu.sync_copy(rows, acc.at[dstv], add=True)
        return carry

    lax.fori_loop(0, nch, body, 0)
    plsc.subcore_barrier()
    pltpu.sync_copy(acc.at[pl.ds(s * rps, rps)],
                    out.at[pl.ds(coff + s * rps, rps)])


def _seg_sum(xsplit, srcr, dstr, dh):
    zeros = jnp.zeros((_NPAD // _NSUB, dh), jnp.float32)
    mesh = plsc.VectorSubcoreMesh(core_axis_name="c", subcore_axis_name="s",
                                  num_cores=_NCORE, num_subcores=_NSUB)
    f = pl.kernel(
        _seg_sum_body,
        out_type=jax.ShapeDtypeStruct((_NCORE * _NPAD, dh), jnp.float32),
        mesh=mesh,
        scratch_types=[
            pltpu.VMEM((_K,), jnp.int32),
            pltpu.VMEM((_K,), jnp.int32),
            pltpu.VMEM((_K,), jnp.int32),
            pltpu.VMEM((_K, dh), jnp.float32),
            pltpu.VMEM_SHARED((_NPAD, dh), jnp.float32),
            pltpu.SemaphoreType.DMA,
        ],
    )
    return f(xsplit, srcr, dstr, zeros)


def _bn_lrelu_split(z, g, bb, m, v, o_ref):
    scale = g * lax.rsqrt(v + 1e-5)
    z = (z - m) * scale + bb
    z = jnp.where(z >= 0, z, 0.01 * z)
    o_ref[0] = z[:, :128]
    o_ref[1] = z[:, 128:]


def _l1_body(x_ref, alo_ref, ahi_ref, ws_ref, wn_ref, b_ref, g_ref, bb_ref,
             m_ref, v_ref, o_ref):
    x = x_ref[...]
    a = jnp.concatenate([alo_ref[:, :64], ahi_ref[:, :64]], axis=1)
    deg = alo_ref[:, 64:65]
    hn = a * (1.0 / jnp.maximum(deg, 1.0))
    z = (jnp.dot(x, ws_ref[...], preferred_element_type=jnp.float32)
         + jnp.dot(hn, wn_ref[...], preferred_element_type=jnp.float32)
         + b_ref[...])
    _bn_lrelu_split(z, g_ref[...], bb_ref[...], m_ref[...], v_ref[...], o_ref)


def _l23_body(xlo_ref, xhi_ref, alo_ref, ahi_ref, d_ref, ws_ref, wn_ref,
              b_ref, g_ref, bb_ref, m_ref, v_ref, o_ref):
    x = jnp.concatenate([xlo_ref[...], xhi_ref[...]], axis=1)
    a = jnp.concatenate([alo_ref[...], ahi_ref[...]], axis=1)
    deg = d_ref[:, 64:65]
    hn = a * (1.0 / jnp.maximum(deg, 1.0))
    z = (jnp.dot(x, ws_ref[...], preferred_element_type=jnp.float32)
         + jnp.dot(hn, wn_ref[...], preferred_element_type=jnp.float32)
         + b_ref[...])
    _bn_lrelu_split(z, g_ref[...], bb_ref[...], m_ref[...], v_ref[...], o_ref)


def _full(shape):
    return pl.BlockSpec(shape, lambda i: (0,) * len(shape))


def _layer1(h_pad, agg1, Ws, Wn, b, g, bb, m, v):
    return pl.pallas_call(
        _l1_body,
        grid=(_NT,),
        in_specs=[
            pl.BlockSpec((_R, 128), lambda i: (i, 0)),
            pl.BlockSpec((_R, 128), lambda i: (i, 0)),
            pl.BlockSpec((_R, 128), lambda i: (_NT + i, 0)),
            _full((128, _H)), _full((128, _H)),
            _full((1, _H)), _full((1, _H)), _full((1, _H)),
            _full((1, _H)), _full((1, _H)),
        ],
        out_specs=pl.BlockSpec((2, _R, 128), lambda i: (0, i, 0)),
        out_shape=jax.ShapeDtypeStruct((2, _NPAD, 128), jnp.float32),
    )(h_pad, agg1, agg1, Ws, Wn, b, g, bb, m, v)


def _layer23(xsplit, agg, agg1, Ws, Wn, b, g, bb, m, v):
    return pl.pallas_call(
        _l23_body,
        grid=(_NT,),
        in_specs=[
            pl.BlockSpec((_R, 128), lambda i: (i, 0)),
            pl.BlockSpec((_R, 128), lambda i: (_NT + i, 0)),
            pl.BlockSpec((_R, 128), lambda i: (i, 0)),
            pl.BlockSpec((_R, 128), lambda i: (_NT + i, 0)),
            pl.BlockSpec((_R, 128), lambda i: (i, 0)),
            _full((_H, _H)), _full((_H, _H)),
            _full((1, _H)), _full((1, _H)), _full((1, _H)),
            _full((1, _H)), _full((1, _H)),
        ],
        out_specs=pl.BlockSpec((2, _R, 128), lambda i: (0, i, 0)),
        out_shape=jax.ShapeDtypeStruct((2, _NPAD, 128), jnp.float32),
    )(xsplit, xsplit, agg, agg, agg1, Ws, Wn, b, g, bb, m, v)


def _final_body(xlo_ref, xhi_ref, f1w_ref, f1b_ref, f2w_ref, f2b_ref,
                f3w_ref, f3b_ref, o_ref, acc_ref):
    i = pl.program_id(0)
    xt = jnp.concatenate([xlo_ref[...], xhi_ref[...]], axis=1)
    row = i * _R + lax.broadcasted_iota(jnp.int32, (_R, 1), 0)
    xt = jnp.where(row < _N, xt, 0.0)

    @pl.when(i == 0)
    def _():
        acc_ref[...] = jnp.zeros_like(acc_ref)

    acc_ref[...] += jnp.sum(xt, axis=0, keepdims=True)

    @pl.when(i == _NT - 1)
    def _():
        hg = acc_ref[...] * (1.0 / _N)
        y = hg @ f1w_ref[...] + f1b_ref[...]
        y = jnp.where(y >= 0, y, 0.01 * y)
        y = y @ f2w_ref[...] + f2b_ref[...]
        y = jnp.where(y >= 0, y, 0.01 * y)
        o_ref[...] = y @ f3w_ref[...] + f3b_ref[...]


def _final(xsplit, f1w, f1b, f2w, f2b, f3w, f3b):
    return pl.pallas_call(
        _final_body,
        grid=(_NT,),
        in_specs=[
            pl.BlockSpec((_R, 128), lambda i: (i, 0)),
            pl.BlockSpec((_R, 128), lambda i: (_NT + i, 0)),
            _full((_H, _H)), _full((1, _H)),
            _full((_H, 1024)), _full((1, 1024)),
            _full((1024, 128)), _full((1, 128)),
        ],
        out_specs=pl.BlockSpec((1, 128), lambda i: (0, 0)),
        out_shape=jax.ShapeDtypeStruct((1, 128), jnp.float32),
        scratch_shapes=[pltpu.VMEM((1, _H), jnp.float32)],
    )(xsplit, xsplit, f1w, f1b, f2w, f2b, f3w, f3b)


def kernel(h, edge_index, Ws1, Wn1, b1, Ws2, Wn2, b2, Ws3, Wn3, b3,
           bn1g, bn1b, bn1m, bn1v, bn2g, bn2b, bn2m, bn2v, bn3g, bn3b,
           bn3m, bn3v, fc1W, fc1b, fc2W, fc2b, fc3W, fc3b):
    f32 = jnp.float32
    src = edge_index[0]
    dst = edge_index[1]

    h_pad = jnp.zeros((_NPAD, 128), f32).at[:_N].set(h)
    onescol = jnp.ones((_NPAD, 1), f32)
    zpad = jnp.zeros((_NPAD, 63), f32)
    zpad2 = jnp.zeros((_NPAD, 64), f32)
    hsplit = jnp.concatenate([
        jnp.concatenate([h_pad[:, :64], onescol, zpad], axis=1),
        jnp.concatenate([h_pad[:, 64:], zpad2], axis=1)], axis=0)

    r1 = lambda a: a.reshape(1, -1)

    agg1 = _seg_sum(hsplit, src, dst, 128)
    x1 = _layer1(h_pad, agg1, Ws1, Wn1, r1(b1), r1(bn1g), r1(bn1b),
                 r1(bn1m), r1(bn1v)).reshape(_NCORE * _NPAD, 128)
    agg2 = _seg_sum(x1, src, dst, 128)
    x2 = _layer23(x1, agg2, agg1, Ws2, Wn2, r1(b2), r1(bn2g), r1(bn2b),
                  r1(bn2m), r1(bn2v)).reshape(_NCORE * _NPAD, 128)
    agg3 = _seg_sum(x2, src, dst, 128)
    x3 = _layer23(x2, agg3, agg1, Ws3, Wn3, r1(b3), r1(bn3g), r1(bn3b),
                  r1(bn3m), r1(bn3v)).reshape(_NCORE * _NPAD, 128)

    f3Wp = jnp.zeros((1024, 128), f32).at[:, :18].set(fc3W)
    f3bp = jnp.zeros((1, 128), f32).at[:, :18].set(r1(fc3b))
    y = _final(x3, fc1W, r1(fc1b), fc2W, r1(fc2b), f3Wp, f3bp)
    return y[:, :18]
